# baseline (speedup 1.0000x reference)
import functools
import math

import jax
import jax.numpy as jnp
from jax import lax
from jax.experimental import pallas as pl
from jax.experimental.pallas import tpu as pltpu
from jax.experimental.pallas import tpu_sc as plsc

_SUPPORT = 1_000_000
_BATCH = 16_384
_NC = 2
_NS = 16
_NW = _NC * _NS
_BPW = _BATCH // _NW
_CHUNK = 128
_NCHUNK = _BPW // _CHUNK

_LSE_BULK = 999_936
_LSE_PAD = 1_000_064

_HALF_LOG_2PI = 0.5 * math.log(2.0 * math.pi)


def _sc_gather(disc, logits, locs, scales):
    mesh = plsc.VectorSubcoreMesh(core_axis_name="c", subcore_axis_name="s")

    @functools.partial(
        pl.kernel,
        mesh=mesh,
        out_type=(jax.ShapeDtypeStruct((_BATCH,), jnp.float32),) * 3,
        scratch_types=[
            pltpu.VMEM((_BPW,), jnp.int32),
            pltpu.VMEM((_BPW,), jnp.float32),
            pltpu.VMEM((_BPW,), jnp.float32),
            pltpu.VMEM((_BPW,), jnp.float32),
            pltpu.SemaphoreType.DMA,
            pltpu.SemaphoreType.DMA,
        ],
    )
    def k(disc_h, logits_h, locs_h, scales_h, glog_h, gloc_h, gscl_h,
          idx_v, a_v, b_v, c_v, gsem, osem):
        wid = lax.axis_index("s") * _NC + lax.axis_index("c")
        base = wid * _BPW
        pltpu.sync_copy(disc_h.at[pl.ds(base, _BPW)], idx_v)
        handles = []
        for j in range(_NCHUNK):
            sl = pl.ds(j * _CHUNK, _CHUNK)
            handles.append(pltpu.async_copy(logits_h.at[idx_v.at[sl]],
                                            a_v.at[sl], gsem))
            handles.append(pltpu.async_copy(locs_h.at[idx_v.at[sl]],
                                            b_v.at[sl], gsem))
            handles.append(pltpu.async_copy(scales_h.at[idx_v.at[sl]],
                                            c_v.at[sl], gsem))
        for h in handles:
            h.wait()
        out = pl.ds(base, _BPW)
        oh = [pltpu.async_copy(a_v, glog_h.at[out], osem),
              pltpu.async_copy(b_v, gloc_h.at[out], osem),
              pltpu.async_copy(c_v, gscl_h.at[out], osem)]
        for h in oh:
            h.wait()

    return k(disc, logits, locs, scales)


def _lse_body(x_hbm, tail_ref, o_ref, x_v, sem):
    cp0 = pltpu.make_async_copy(x_hbm.at[pl.ds(0, _LSE_BULK)],
                                x_v.at[pl.ds(0, _LSE_BULK)], sem)
    cp0.start()
    x_v[pl.ds(_LSE_BULK, 128)] = tail_ref[...]
    cp0.wait()
    v = x_v[...]
    m = jnp.max(v)
    o_ref[0] = m + jnp.log(jnp.sum(jnp.exp(v - m)))


def _lse(logits):
    tail = jnp.concatenate(
        [logits[_LSE_BULK:], jnp.full((128 - (_SUPPORT - _LSE_BULK),),
                                      -1e30, jnp.float32)])
    return pl.pallas_call(
        _lse_body,
        out_shape=jax.ShapeDtypeStruct((1,), jnp.float32),
        in_specs=[pl.BlockSpec(memory_space=pl.ANY),
                  pl.BlockSpec(memory_space=pltpu.VMEM)],
        out_specs=pl.BlockSpec(memory_space=pltpu.SMEM),
        scratch_shapes=[pltpu.VMEM((_LSE_PAD,), jnp.float32),
                        pltpu.SemaphoreType.DMA],
    )(logits, tail)


def _combine_body(logz_ref, glog_ref, gloc_ref, gscl_ref, cont_ref, o_ref):
    z = (cont_ref[...] - gloc_ref[...]) / gscl_ref[...]
    o_ref[...] = (glog_ref[...] - logz_ref[0] - 0.5 * z * z
                  - jnp.log(gscl_ref[...]) - _HALF_LOG_2PI)


def _combine(logz, glog, gloc, gscl, cont):
    return pl.pallas_call(
        _combine_body,
        out_shape=jax.ShapeDtypeStruct((_BATCH,), jnp.float32),
        in_specs=[pl.BlockSpec(memory_space=pltpu.SMEM)]
                 + [pl.BlockSpec(memory_space=pltpu.VMEM)] * 4,
        out_specs=pl.BlockSpec(memory_space=pltpu.VMEM),
    )(logz, glog, gloc, gscl, cont)


def kernel(discrete, continuous, logits, locs, scales):
    disc = discrete.astype(jnp.int32)
    glog, gloc, gscl = _sc_gather(disc, logits, locs, scales)
    logz = _lse(logits)
    return _combine(logz, glog, gloc, gscl, continuous)


# speedup vs baseline: 1.1542x; 1.1542x over previous
import functools
import math

import jax
import jax.numpy as jnp
from jax import lax
from jax.experimental import pallas as pl
from jax.experimental.pallas import tpu as pltpu
from jax.experimental.pallas import tpu_sc as plsc

_SUPPORT = 1_000_000
_BATCH = 16_384
_NC = 2
_NS = 16
_NW = _NC * _NS
_BPW = _BATCH // _NW
_CHUNK = 128
_NCHUNK = _BPW // _CHUNK

_LSE_BULK = 999_936
_LSE_PAD = 1_000_064

_HALF_LOG_2PI = 0.5 * math.log(2.0 * math.pi)


def _sc_gather(disc, logits, locs, scales):
    mesh = plsc.VectorSubcoreMesh(core_axis_name="c", subcore_axis_name="s")

    @functools.partial(
        pl.kernel,
        mesh=mesh,
        out_type=(jax.ShapeDtypeStruct((_BATCH,), jnp.float32),) * 3,
        scratch_types=[
            pltpu.VMEM((_BPW,), jnp.int32),
            pltpu.VMEM((_BPW,), jnp.float32),
            pltpu.VMEM((_BPW,), jnp.float32),
            pltpu.VMEM((_BPW,), jnp.float32),
            pltpu.SemaphoreType.DMA,
            pltpu.SemaphoreType.DMA,
        ],
    )
    def k(disc_h, logits_h, locs_h, scales_h, glog_h, gloc_h, gscl_h,
          idx_v, a_v, b_v, c_v, gsem, osem):
        wid = lax.axis_index("s") * _NC + lax.axis_index("c")
        base = wid * _BPW
        pltpu.sync_copy(disc_h.at[pl.ds(base, _BPW)], idx_v)
        handles = []
        for j in range(_NCHUNK):
            sl = pl.ds(j * _CHUNK, _CHUNK)
            handles.append(pltpu.async_copy(logits_h.at[idx_v.at[sl]],
                                            a_v.at[sl], gsem))
            handles.append(pltpu.async_copy(locs_h.at[idx_v.at[sl]],
                                            b_v.at[sl], gsem))
            handles.append(pltpu.async_copy(scales_h.at[idx_v.at[sl]],
                                            c_v.at[sl], gsem))
        for h in handles:
            h.wait()
        out = pl.ds(base, _BPW)
        oh = [pltpu.async_copy(a_v, glog_h.at[out], osem),
              pltpu.async_copy(b_v, gloc_h.at[out], osem),
              pltpu.async_copy(c_v, gscl_h.at[out], osem)]
        for h in oh:
            h.wait()

    return k(disc, logits, locs, scales)


def _lse_body(x_ref, tail_ref, o_ref):
    v = x_ref[...]
    t = tail_ref[...]
    m = jnp.maximum(jnp.max(v), jnp.max(t))
    s = jnp.sum(jnp.exp(v - m)) + jnp.sum(jnp.exp(t - m))
    o_ref[0] = m + jnp.log(s)


def _lse(logits):
    bulk = logits[:_LSE_BULK].reshape(_LSE_BULK // 128, 128)
    tail = jnp.concatenate(
        [logits[_LSE_BULK:], jnp.full((128 - (_SUPPORT - _LSE_BULK),),
                                      -1e30, jnp.float32)])
    return pl.pallas_call(
        _lse_body,
        out_shape=jax.ShapeDtypeStruct((1,), jnp.float32),
        in_specs=[pl.BlockSpec(memory_space=pltpu.VMEM),
                  pl.BlockSpec(memory_space=pltpu.VMEM)],
        out_specs=pl.BlockSpec(memory_space=pltpu.SMEM),
    )(bulk, tail)


def _combine_body(logz_ref, glog_ref, gloc_ref, gscl_ref, cont_ref, o_ref):
    z = (cont_ref[...] - gloc_ref[...]) / gscl_ref[...]
    o_ref[...] = (glog_ref[...] - logz_ref[0] - 0.5 * z * z
                  - jnp.log(gscl_ref[...]) - _HALF_LOG_2PI)


def _combine(logz, glog, gloc, gscl, cont):
    return pl.pallas_call(
        _combine_body,
        out_shape=jax.ShapeDtypeStruct((_BATCH,), jnp.float32),
        in_specs=[pl.BlockSpec(memory_space=pltpu.SMEM)]
                 + [pl.BlockSpec(memory_space=pltpu.VMEM)] * 4,
        out_specs=pl.BlockSpec(memory_space=pltpu.VMEM),
    )(logz, glog, gloc, gscl, cont)


def kernel(discrete, continuous, logits, locs, scales):
    disc = discrete.astype(jnp.int32)
    glog, gloc, gscl = _sc_gather(disc, logits, locs, scales)
    logz = _lse(logits)
    return _combine(logz, glog, gloc, gscl, continuous)


# speedup vs baseline: 1.3144x; 1.1388x over previous
import functools
import math

import jax
import jax.numpy as jnp
from jax import lax
from jax.experimental import pallas as pl
from jax.experimental.pallas import tpu as pltpu
from jax.experimental.pallas import tpu_sc as plsc

_SUPPORT = 1_000_000
_BATCH = 16_384
_NC = 2
_NS = 16
_NW = _NC * _NS
_BPW = _BATCH // _NW
_CHUNK = 128
_NCHUNK = _BPW // _CHUNK

_LSE_BULK = 999_936
_LSE_PAD = 1_000_064

_HALF_LOG_2PI = 0.5 * math.log(2.0 * math.pi)


def _sc_gather(disc, logits, locs, scales):
    mesh = plsc.VectorSubcoreMesh(core_axis_name="c", subcore_axis_name="s")

    @functools.partial(
        pl.kernel,
        mesh=mesh,
        out_type=(jax.ShapeDtypeStruct((_BATCH,), jnp.float32),) * 3,
        scratch_types=[
            pltpu.VMEM((_BPW,), jnp.int32),
            pltpu.VMEM((_BPW,), jnp.float32),
            pltpu.VMEM((_BPW,), jnp.float32),
            pltpu.VMEM((_BPW,), jnp.float32),
            pltpu.SemaphoreType.DMA,
            pltpu.SemaphoreType.DMA,
        ],
    )
    def k(disc_h, logits_h, locs_h, scales_h, glog_h, gloc_h, gscl_h,
          idx_v, a_v, b_v, c_v, gsem, osem):
        wid = lax.axis_index("s") * _NC + lax.axis_index("c")
        base = wid * _BPW
        pltpu.sync_copy(disc_h.at[pl.ds(base, _BPW)], idx_v)
        handles = []
        for j in range(_NCHUNK):
            sl = pl.ds(j * _CHUNK, _CHUNK)
            handles.append(pltpu.async_copy(logits_h.at[idx_v.at[sl]],
                                            a_v.at[sl], gsem))
            handles.append(pltpu.async_copy(locs_h.at[idx_v.at[sl]],
                                            b_v.at[sl], gsem))
            handles.append(pltpu.async_copy(scales_h.at[idx_v.at[sl]],
                                            c_v.at[sl], gsem))
        for h in handles:
            h.wait()
        out = pl.ds(base, _BPW)
        oh = [pltpu.async_copy(a_v, glog_h.at[out], osem),
              pltpu.async_copy(b_v, gloc_h.at[out], osem),
              pltpu.async_copy(c_v, gscl_h.at[out], osem)]
        for h in oh:
            h.wait()

    return k(disc, logits, locs, scales)


_NDMA = 4
_DMA_CHUNK = _LSE_BULK // _NDMA


def _lse_body(x_hbm, tail_ref, o_ref, x_v, sem):
    cps = []
    for i in range(_NDMA):
        sl = pl.ds(i * _DMA_CHUNK, _DMA_CHUNK)
        cps.append(pltpu.make_async_copy(x_hbm.at[sl], x_v.at[sl], sem))
    for cp in cps:
        cp.start()
    x_v[pl.ds(_LSE_BULK, 128)] = tail_ref[...]
    for cp in cps:
        cp.wait()
    v = pltpu.einshape("(ab)->ab", x_v[...], b=128)
    m = jnp.max(v)
    o_ref[0] = m + jnp.log(jnp.sum(jnp.exp(v - m)))


def _lse(logits):
    tail = jnp.concatenate(
        [logits[_LSE_BULK:], jnp.full((128 - (_SUPPORT - _LSE_BULK),),
                                      -1e30, jnp.float32)])
    return pl.pallas_call(
        _lse_body,
        out_shape=jax.ShapeDtypeStruct((1,), jnp.float32),
        in_specs=[pl.BlockSpec(memory_space=pl.ANY),
                  pl.BlockSpec(memory_space=pltpu.VMEM)],
        out_specs=pl.BlockSpec(memory_space=pltpu.SMEM),
        scratch_shapes=[pltpu.VMEM((_LSE_PAD,), jnp.float32),
                        pltpu.SemaphoreType.DMA],
    )(logits, tail)


def _combine_body(logz_ref, glog_ref, gloc_ref, gscl_ref, cont_ref, o_ref):
    z = (cont_ref[...] - gloc_ref[...]) / gscl_ref[...]
    o_ref[...] = (glog_ref[...] - logz_ref[0] - 0.5 * z * z
                  - jnp.log(gscl_ref[...]) - _HALF_LOG_2PI)


def _combine(logz, glog, gloc, gscl, cont):
    return pl.pallas_call(
        _combine_body,
        out_shape=jax.ShapeDtypeStruct((_BATCH,), jnp.float32),
        in_specs=[pl.BlockSpec(memory_space=pltpu.SMEM)]
                 + [pl.BlockSpec(memory_space=pltpu.VMEM)] * 4,
        out_specs=pl.BlockSpec(memory_space=pltpu.VMEM),
    )(logz, glog, gloc, gscl, cont)


def kernel(discrete, continuous, logits, locs, scales):
    disc = discrete.astype(jnp.int32)
    glog, gloc, gscl = _sc_gather(disc, logits, locs, scales)
    logz = _lse(logits)
    return _combine(logz, glog, gloc, gscl, continuous)
